# baseline (speedup 1.0000x reference)
import functools

import jax
import jax.numpy as jnp
from jax import lax
from jax.experimental import pallas as pl
from jax.experimental.pallas import tpu as pltpu
from jax.experimental.pallas import tpu_sc as plsc

TOPK = 10
ROWS_PER_BLOCK = 16
LANES = 128


def _naive_topk_v(x, num_cols):
    R, N = x.shape
    lane = jax.lax.broadcasted_iota(jnp.int32, (R, N), dimension=1)
    ciota = jax.lax.broadcasted_iota(jnp.int32, (R, num_cols), dimension=1)
    neg_inf = jnp.float32(-jnp.inf)
    big = jnp.int32(N)
    V = jnp.zeros((R, num_cols), jnp.float32)
    y = x
    for j in range(TOPK):
        m = jnp.max(y, axis=1, keepdims=True)
        V = jnp.where(ciota == j, m, V)
        if j < TOPK - 1:
            key = jnp.where(y == m, lane, big)
            fi = jnp.min(key, axis=1, keepdims=True)
            y = jnp.where(lane == fi, neg_inf, y)
    return V


def _fused_kernel(x_ref, W0_ref, b0_ref, W1_ref, b1_ref, W2_ref, b2_ref,
                  o_ref, vals_ref):
    R, N = x_ref.shape
    neg_inf = jnp.float32(-jnp.inf)
    W = LANES
    TILE = 1024

    def merge4(s1, s2):
        a1, b1, c1, d1 = s1
        a2, b2, c2, d2 = s2
        m1 = jnp.maximum(a1, d2)
        m2 = jnp.maximum(b1, c2)
        m3 = jnp.maximum(c1, b2)
        m4 = jnp.maximum(d1, a2)
        a = jnp.maximum(m1, m3)
        c = jnp.minimum(m1, m3)
        b = jnp.maximum(m2, m4)
        d = jnp.minimum(m2, m4)
        return (jnp.maximum(a, b), jnp.minimum(a, b),
                jnp.maximum(c, d), jnp.minimum(c, d))

    state = None
    for tb in range(N // TILE):
        base = tb * TILE
        w = TILE // 2
        xa = x_ref[:, base:base + w]
        xb = x_ref[:, base + w:base + TILE]
        A = jnp.maximum(xa, xb)
        B = jnp.minimum(xa, xb)
        w //= 2
        a1, a2 = A[:, :w], A[:, w:]
        b1, b2 = B[:, :w], B[:, w:]
        o1 = jnp.maximum(a1, a2)
        t = jnp.minimum(a1, a2)
        u = jnp.maximum(b1, b2)
        o4 = jnp.minimum(b1, b2)
        s = (o1, jnp.maximum(t, u), jnp.minimum(t, u), o4)
        while w > W:
            w //= 2
            s = merge4(tuple(v[:, :w] for v in s),
                       tuple(v[:, w:] for v in s))
        state = s if state is None else merge4(state, s)
    A, B, C, D = state

    wiota = jax.lax.broadcasted_iota(jnp.int32, (R, W), dimension=1)
    pops = jnp.zeros((R, W), jnp.int32)
    V = jnp.zeros((R, W), jnp.float32)
    for j in range(TOPK):
        m = jnp.max(A, axis=1, keepdims=True)
        V = jnp.where(wiota == j, m, V)
        l0 = jnp.min(jnp.where(A == m, wiota, W), axis=1, keepdims=True)
        oh = wiota == l0
        A = jnp.where(oh, B, A)
        B = jnp.where(oh, C, B)
        C = jnp.where(oh, D, C)
        D = jnp.where(oh, neg_inf, D)
        pops = pops + oh.astype(jnp.int32)
    vals_ref[...] = V

    @pl.when(jnp.max(pops) >= 4)
    def _():
        vals_ref[...] = _naive_topk_v(x_ref[...], W)

    Vv = vals_ref[...]

    def bf(v):
        return v.astype(jnp.bfloat16).astype(jnp.float32)

    W0b, W1b, W2b = bf(W0_ref[...]), bf(W1_ref[...]), bf(W2_ref[...])
    Vb = bf(Vv)
    h = b0_ref[...]
    for i in range(TOPK):
        h = h + Vb[:, i:i + 1] * W0b[i:i + 1, :]
    h = jnp.maximum(h, 0.0)
    hb = bf(h)
    g = b1_ref[...]
    for i in range(5):
        g = g + hb[:, i:i + 1] * W1b[i:i + 1, :]
    g = jnp.maximum(g, 0.0)
    gb = bf(g)
    t = b2_ref[...]
    for i in range(5):
        t = t + gb[:, i:i + 1] * W2b[i:i + 1, :]
    tau = jnp.clip(jnp.abs(t), 1e-12, 1e12)

    inv_tau = 1.0 / tau
    m0 = Vv[:, 0:1]
    s_acc = jnp.zeros((R, TILE), jnp.float32)
    for tb in range(N // TILE):
        base = tb * TILE
        e = jnp.exp((x_ref[:, base:base + TILE] - m0) * inv_tau)
        o_ref[:, base:base + TILE] = e
        s_acc = s_acc + e
    inv_s = 1.0 / jnp.sum(s_acc, axis=1, keepdims=True)
    for tb in range(N // TILE):
        base = tb * TILE
        o_ref[:, base:base + TILE] = o_ref[:, base:base + TILE] * inv_s


def _sc_topk_build(B, N):
    mesh = plsc.VectorSubcoreMesh(core_axis_name="c", subcore_axis_name="s")
    NC, NS = mesh.num_cores, mesh.num_subcores
    NW = NC * NS
    rows_per_w = B // NW
    neg_inf = jnp.float32(-jnp.inf)

    @functools.partial(
        pl.kernel, mesh=mesh,
        out_type=jax.ShapeDtypeStruct((B, 16), jnp.float32),
        scratch_types=[
            pltpu.VMEM((N,), jnp.float32),
            pltpu.VMEM((16,), jnp.float32),
        ],
    )
    def sc_topk(x_hbm, out_hbm, row_v, out_v):
        wid = lax.axis_index("s") * NC + lax.axis_index("c")
        iota = lax.iota(jnp.int32, 16)

        def bsplat_max(v):
            for d in (1, 2, 4, 8):
                v = jnp.maximum(v, v[iota ^ d])
            return v

        def first_onehot(eq):
            e = jnp.where(eq, jnp.int32(1), jnp.int32(0))
            p = e
            for d in (1, 2, 4, 8):
                sh = jnp.where(iota >= d, p[jnp.maximum(iota - d, 0)],
                               jnp.int32(0))
                p = p | sh
            excl = jnp.where(iota >= 1, p[jnp.maximum(iota - 1, 0)],
                             jnp.int32(0))
            return eq & (excl == 0)

        for rr in range(rows_per_w):
            row = wid * rows_per_w + rr
            pltpu.sync_copy(x_hbm.at[row], row_v)

            def body(i, st):
                t = row_v[pl.ds(i * 16, 16)]
                out = []
                for k in range(TOPK):
                    hi = jnp.maximum(st[k], t)
                    t = jnp.minimum(st[k], t)
                    out.append(hi)
                return tuple(out)

            init = tuple(jnp.full((16,), neg_inf) for _ in range(TOPK))
            st = lax.fori_loop(0, N // 16, body, init, unroll=4)

            OUT = jnp.full((16,), neg_inf)
            stack = list(st)
            for j in range(TOPK):
                mv = bsplat_max(stack[0])
                OUT = jnp.where(iota == j, mv, OUT)
                oh = first_onehot(stack[0] == mv)
                for k in range(TOPK - 1):
                    stack[k] = jnp.where(oh, stack[k + 1], stack[k])
                stack[TOPK - 1] = jnp.where(oh, neg_inf, stack[TOPK - 1])
            out_v[...] = OUT
            pltpu.sync_copy(out_v, out_hbm.at[row])

    return sc_topk


def _mlp_softmax_kernel(x_ref, v_ref, W0_ref, b0_ref, W1_ref, b1_ref,
                        W2_ref, b2_ref, o_ref):
    R, N = x_ref.shape
    TILE = 1024
    Vv = v_ref[...]

    def bf(v):
        return v.astype(jnp.bfloat16).astype(jnp.float32)

    W0b, W1b, W2b = bf(W0_ref[...]), bf(W1_ref[...]), bf(W2_ref[...])
    Vb = bf(Vv)
    h = b0_ref[...]
    for i in range(TOPK):
        h = h + Vb[:, i:i + 1] * W0b[i:i + 1, :]
    h = jnp.maximum(h, 0.0)
    hb = bf(h)
    g = b1_ref[...]
    for i in range(5):
        g = g + hb[:, i:i + 1] * W1b[i:i + 1, :]
    g = jnp.maximum(g, 0.0)
    gb = bf(g)
    t = b2_ref[...]
    for i in range(5):
        t = t + gb[:, i:i + 1] * W2b[i:i + 1, :]
    tau = jnp.clip(jnp.abs(t), 1e-12, 1e12)

    inv_tau = 1.0 / tau
    m0 = Vv[:, 0:1]
    s_acc = jnp.zeros((R, TILE), jnp.float32)
    for tb in range(N // TILE):
        base = tb * TILE
        e = jnp.exp((x_ref[:, base:base + TILE] - m0) * inv_tau)
        o_ref[:, base:base + TILE] = e
        s_acc = s_acc + e
    inv_s = 1.0 / jnp.sum(s_acc, axis=1, keepdims=True)
    for tb in range(N // TILE):
        base = tb * TILE
        o_ref[:, base:base + TILE] = o_ref[:, base:base + TILE] * inv_s


@jax.jit
def kernel(inp, W0, b0, W1, b1, W2, b2):
    B, N = inp.shape
    R = ROWS_PER_BLOCK
    vals = _sc_topk_build(B, N)(inp)
    full = lambda i: (0, 0)
    return pl.pallas_call(
        _mlp_softmax_kernel,
        grid=(B // R,),
        in_specs=[
            pl.BlockSpec((R, N), lambda i: (i, 0)),
            pl.BlockSpec((R, 16), lambda i: (i, 0)),
            pl.BlockSpec((TOPK, 5), full),
            pl.BlockSpec((1, 5), full),
            pl.BlockSpec((5, 5), full),
            pl.BlockSpec((1, 5), full),
            pl.BlockSpec((5, 1), full),
            pl.BlockSpec((1, 1), full),
        ],
        out_specs=pl.BlockSpec((R, N), lambda i: (i, 0)),
        out_shape=jax.ShapeDtypeStruct((B, N), jnp.float32),
    )(inp, vals, W0, b0.reshape(1, 5), W1, b1.reshape(1, 5), W2,
      b2.reshape(1, 1))


@jax.jit
def _kernel_tc_fused(inp, W0, b0, W1, b1, W2, b2):
    B, N = inp.shape
    R = ROWS_PER_BLOCK
    grid = (B // R,)
    full = lambda i: (0, 0)
    out = pl.pallas_call(
        _fused_kernel,
        grid=grid,
        in_specs=[
            pl.BlockSpec((R, N), lambda i: (i, 0)),
            pl.BlockSpec((TOPK, 5), full),
            pl.BlockSpec((1, 5), full),
            pl.BlockSpec((5, 5), full),
            pl.BlockSpec((1, 5), full),
            pl.BlockSpec((5, 1), full),
            pl.BlockSpec((1, 1), full),
        ],
        out_specs=pl.BlockSpec((R, N), lambda i: (i, 0)),
        out_shape=jax.ShapeDtypeStruct((B, N), jnp.float32),
        scratch_shapes=[pltpu.VMEM((R, LANES), jnp.float32)],
    )(inp, W0, b0.reshape(1, 5), W1, b1.reshape(1, 5), W2, b2.reshape(1, 1))
    return out


# speedup vs baseline: 3.0095x; 3.0095x over previous
import jax
import jax.numpy as jnp
from jax.experimental import pallas as pl
from jax.experimental.pallas import tpu as pltpu

TOPK = 10
ROWS_PER_BLOCK = 32
LANES = 128


def _naive_topk_v(x, num_cols):
    R, N = x.shape
    lane = jax.lax.broadcasted_iota(jnp.int32, (R, N), dimension=1)
    ciota = jax.lax.broadcasted_iota(jnp.int32, (R, num_cols), dimension=1)
    neg_inf = jnp.float32(-jnp.inf)
    big = jnp.int32(N)
    V = jnp.zeros((R, num_cols), jnp.float32)
    y = x
    for j in range(TOPK):
        m = jnp.max(y, axis=1, keepdims=True)
        V = jnp.where(ciota == j, m, V)
        if j < TOPK - 1:
            key = jnp.where(y == m, lane, big)
            fi = jnp.min(key, axis=1, keepdims=True)
            y = jnp.where(lane == fi, neg_inf, y)
    return V


def _fused_kernel(x_ref, W0_ref, b0_ref, W1_ref, b1_ref, W2_ref, b2_ref,
                  o_ref, vals_ref):
    R, N = x_ref.shape
    neg_inf = jnp.float32(-jnp.inf)
    W = LANES
    TILE = 1024

    def merge4(s1, s2):
        a1, b1, c1, d1 = s1
        a2, b2, c2, d2 = s2
        m1 = jnp.maximum(a1, d2)
        m2 = jnp.maximum(b1, c2)
        m3 = jnp.maximum(c1, b2)
        m4 = jnp.maximum(d1, a2)
        a = jnp.maximum(m1, m3)
        c = jnp.minimum(m1, m3)
        b = jnp.maximum(m2, m4)
        d = jnp.minimum(m2, m4)
        return (jnp.maximum(a, b), jnp.minimum(a, b),
                jnp.maximum(c, d), jnp.minimum(c, d))

    state = None
    for tb in range(N // TILE):
        base = tb * TILE
        w = TILE // 2
        xa = x_ref[:, base:base + w]
        xb = x_ref[:, base + w:base + TILE]
        A = jnp.maximum(xa, xb)
        B = jnp.minimum(xa, xb)
        w //= 2
        a1, a2 = A[:, :w], A[:, w:]
        b1, b2 = B[:, :w], B[:, w:]
        o1 = jnp.maximum(a1, a2)
        t = jnp.minimum(a1, a2)
        u = jnp.maximum(b1, b2)
        o4 = jnp.minimum(b1, b2)
        s = (o1, jnp.maximum(t, u), jnp.minimum(t, u), o4)
        while w > W:
            w //= 2
            s = merge4(tuple(v[:, :w] for v in s),
                       tuple(v[:, w:] for v in s))
        state = s if state is None else merge4(state, s)
    A, B, C, D = state

    wiota = jax.lax.broadcasted_iota(jnp.int32, (R, W), dimension=1)
    pops = jnp.zeros((R, W), jnp.int32)
    V = jnp.zeros((R, W), jnp.float32)
    for j in range(TOPK):
        m = jnp.max(A, axis=1, keepdims=True)
        V = jnp.where(wiota == j, m, V)
        l0 = jnp.min(jnp.where(A == m, wiota, W), axis=1, keepdims=True)
        oh = wiota == l0
        A = jnp.where(oh, B, A)
        B = jnp.where(oh, C, B)
        C = jnp.where(oh, D, C)
        D = jnp.where(oh, neg_inf, D)
        pops = pops + oh.astype(jnp.int32)
    vals_ref[...] = V

    @pl.when(jnp.max(pops) >= 4)
    def _():
        vals_ref[...] = _naive_topk_v(x_ref[...], W)

    Vv = vals_ref[...]

    def bf(v):
        return v.astype(jnp.bfloat16).astype(jnp.float32)

    W0b, W1b, W2b = bf(W0_ref[...]), bf(W1_ref[...]), bf(W2_ref[...])
    Vb = bf(Vv)
    h = b0_ref[...]
    for i in range(TOPK):
        h = h + Vb[:, i:i + 1] * W0b[i:i + 1, :]
    h = jnp.maximum(h, 0.0)
    hb = bf(h)
    g = b1_ref[...]
    for i in range(5):
        g = g + hb[:, i:i + 1] * W1b[i:i + 1, :]
    g = jnp.maximum(g, 0.0)
    gb = bf(g)
    t = b2_ref[...]
    for i in range(5):
        t = t + gb[:, i:i + 1] * W2b[i:i + 1, :]
    tau = jnp.clip(jnp.abs(t), 1e-12, 1e12)

    inv_tau = 1.0 / tau
    m0 = Vv[:, 0:1]
    s_acc = jnp.zeros((R, TILE), jnp.float32)
    for tb in range(N // TILE):
        base = tb * TILE
        e = jnp.exp((x_ref[:, base:base + TILE] - m0) * inv_tau)
        o_ref[:, base:base + TILE] = e
        s_acc = s_acc + e
    inv_s = 1.0 / jnp.sum(s_acc, axis=1, keepdims=True)
    for tb in range(N // TILE):
        base = tb * TILE
        o_ref[:, base:base + TILE] = o_ref[:, base:base + TILE] * inv_s


@jax.jit
def kernel(inp, W0, b0, W1, b1, W2, b2):
    B, N = inp.shape
    R = ROWS_PER_BLOCK
    grid = (B // R,)
    full = lambda i: (0, 0)
    out = pl.pallas_call(
        _fused_kernel,
        grid=grid,
        in_specs=[
            pl.BlockSpec((R, N), lambda i: (i, 0)),
            pl.BlockSpec((TOPK, 5), full),
            pl.BlockSpec((1, 5), full),
            pl.BlockSpec((5, 5), full),
            pl.BlockSpec((1, 5), full),
            pl.BlockSpec((5, 1), full),
            pl.BlockSpec((1, 1), full),
        ],
        out_specs=pl.BlockSpec((R, N), lambda i: (i, 0)),
        out_shape=jax.ShapeDtypeStruct((B, N), jnp.float32),
        scratch_shapes=[pltpu.VMEM((R, LANES), jnp.float32)],
    )(inp, W0, b0.reshape(1, 5), W1, b1.reshape(1, 5), W2, b2.reshape(1, 1))
    return out


# speedup vs baseline: 3.4276x; 1.1389x over previous
import jax
import jax.numpy as jnp
from jax.experimental import pallas as pl
from jax.experimental.pallas import tpu as pltpu

TOPK = 10
ROWS_PER_BLOCK = 64
LANES = 128


def _naive_topk_v(x, num_cols):
    R, N = x.shape
    lane = jax.lax.broadcasted_iota(jnp.int32, (R, N), dimension=1)
    ciota = jax.lax.broadcasted_iota(jnp.int32, (R, num_cols), dimension=1)
    neg_inf = jnp.float32(-jnp.inf)
    big = jnp.int32(N)
    V = jnp.zeros((R, num_cols), jnp.float32)
    y = x
    for j in range(TOPK):
        m = jnp.max(y, axis=1, keepdims=True)
        V = jnp.where(ciota == j, m, V)
        if j < TOPK - 1:
            key = jnp.where(y == m, lane, big)
            fi = jnp.min(key, axis=1, keepdims=True)
            y = jnp.where(lane == fi, neg_inf, y)
    return V


def _fused_kernel(x_ref, W0_ref, b0_ref, W1_ref, b1_ref, W2_ref, b2_ref,
                  o_ref, vals_ref):
    R, N = x_ref.shape
    neg_inf = jnp.float32(-jnp.inf)
    W = LANES
    TILE = 1024

    def merge4(s1, s2):
        a1, b1, c1, d1 = s1
        a2, b2, c2, d2 = s2
        m1 = jnp.maximum(a1, d2)
        m2 = jnp.maximum(b1, c2)
        m3 = jnp.maximum(c1, b2)
        m4 = jnp.maximum(d1, a2)
        a = jnp.maximum(m1, m3)
        c = jnp.minimum(m1, m3)
        b = jnp.maximum(m2, m4)
        d = jnp.minimum(m2, m4)
        return (jnp.maximum(a, b), jnp.minimum(a, b),
                jnp.maximum(c, d), jnp.minimum(c, d))

    state = None
    for tb in range(N // TILE):
        base = tb * TILE
        w = TILE // 2
        xa = x_ref[:, base:base + w]
        xb = x_ref[:, base + w:base + TILE]
        A = jnp.maximum(xa, xb)
        B = jnp.minimum(xa, xb)
        w //= 2
        a1, a2 = A[:, :w], A[:, w:]
        b1, b2 = B[:, :w], B[:, w:]
        o1 = jnp.maximum(a1, a2)
        t = jnp.minimum(a1, a2)
        u = jnp.maximum(b1, b2)
        o4 = jnp.minimum(b1, b2)
        s = (o1, jnp.maximum(t, u), jnp.minimum(t, u), o4)
        while w > W:
            w //= 2
            s = merge4(tuple(v[:, :w] for v in s),
                       tuple(v[:, w:] for v in s))
        state = s if state is None else merge4(state, s)
    A, B, C, D = state

    wiota = jax.lax.broadcasted_iota(jnp.int32, (R, W), dimension=1)
    pops = jnp.zeros((R, W), jnp.int32)
    V = jnp.zeros((R, W), jnp.float32)
    for j in range(TOPK):
        m = jnp.max(A, axis=1, keepdims=True)
        V = jnp.where(wiota == j, m, V)
        l0 = jnp.min(jnp.where(A == m, wiota, W), axis=1, keepdims=True)
        oh = wiota == l0
        A = jnp.where(oh, B, A)
        B = jnp.where(oh, C, B)
        C = jnp.where(oh, D, C)
        D = jnp.where(oh, neg_inf, D)
        pops = pops + oh.astype(jnp.int32)
    vals_ref[...] = V

    @pl.when(jnp.max(pops) >= 4)
    def _():
        vals_ref[...] = _naive_topk_v(x_ref[...], W)

    Vv = vals_ref[...]

    def bf(v):
        return v.astype(jnp.bfloat16).astype(jnp.float32)

    W0b, W1b, W2b = bf(W0_ref[...]), bf(W1_ref[...]), bf(W2_ref[...])
    Vb = bf(Vv)
    h = b0_ref[...]
    for i in range(TOPK):
        h = h + Vb[:, i:i + 1] * W0b[i:i + 1, :]
    h = jnp.maximum(h, 0.0)
    hb = bf(h)
    g = b1_ref[...]
    for i in range(5):
        g = g + hb[:, i:i + 1] * W1b[i:i + 1, :]
    g = jnp.maximum(g, 0.0)
    gb = bf(g)
    t = b2_ref[...]
    for i in range(5):
        t = t + gb[:, i:i + 1] * W2b[i:i + 1, :]
    tau = jnp.clip(jnp.abs(t), 1e-12, 1e12)

    inv_tau = 1.0 / tau
    m0 = Vv[:, 0:1]
    s_acc = jnp.zeros((R, TILE), jnp.float32)
    for tb in range(N // TILE):
        base = tb * TILE
        e = jnp.exp((x_ref[:, base:base + TILE] - m0) * inv_tau)
        o_ref[:, base:base + TILE] = e
        s_acc = s_acc + e
    inv_s = 1.0 / jnp.sum(s_acc, axis=1, keepdims=True)
    for tb in range(N // TILE):
        base = tb * TILE
        o_ref[:, base:base + TILE] = o_ref[:, base:base + TILE] * inv_s


@jax.jit
def kernel(inp, W0, b0, W1, b1, W2, b2):
    B, N = inp.shape
    R = ROWS_PER_BLOCK
    grid = (B // R,)
    full = lambda i: (0, 0)
    out = pl.pallas_call(
        _fused_kernel,
        grid=grid,
        in_specs=[
            pl.BlockSpec((R, N), lambda i: (i, 0)),
            pl.BlockSpec((TOPK, 5), full),
            pl.BlockSpec((1, 5), full),
            pl.BlockSpec((5, 5), full),
            pl.BlockSpec((1, 5), full),
            pl.BlockSpec((5, 1), full),
            pl.BlockSpec((1, 1), full),
        ],
        out_specs=pl.BlockSpec((R, N), lambda i: (i, 0)),
        out_shape=jax.ShapeDtypeStruct((B, N), jnp.float32),
        scratch_shapes=[pltpu.VMEM((R, LANES), jnp.float32)],
    )(inp, W0, b0.reshape(1, 5), W1, b1.reshape(1, 5), W2, b2.reshape(1, 1))
    return out
